# baseline (speedup 1.0000x reference)
import functools

import jax
import jax.numpy as jnp
from jax import lax
from jax.experimental import pallas as pl
from jax.experimental.pallas import tpu as pltpu
from jax.experimental.pallas import tpu_sc as plsc

_EMB = 32
_CHUNK = 128
_NC, _NS, _NL = 2, 16, 16


@functools.lru_cache(maxsize=None)
def _recnet_sc(batch):
    n_workers = _NC * _NS
    bpw = batch // n_workers
    n_chunks = bpw // _CHUNK
    n_groups = bpw // _NL
    mesh = plsc.VectorSubcoreMesh(
        core_axis_name="c", subcore_axis_name="s",
        num_cores=_NC, num_subcores=_NS)

    @functools.partial(
        pl.kernel,
        mesh=mesh,
        out_type=jax.ShapeDtypeStruct((batch,), jnp.float32),
        compiler_params=pltpu.CompilerParams(
            needs_layout_passes=False, use_tc_tiling_on_sc=False),
        scratch_types=[
            pltpu.VMEM((n_chunks, _CHUNK), jnp.int32),
            pltpu.VMEM((n_chunks, _CHUNK), jnp.int32),
            pltpu.VMEM((bpw, _EMB), jnp.float32),
            pltpu.VMEM((bpw, _EMB), jnp.float32),
            pltpu.VMEM((bpw,), jnp.float32),
            pltpu.VMEM((bpw,), jnp.float32),
            pltpu.VMEM((bpw,), jnp.float32),
            pltpu.SemaphoreType.DMA,
        ],
    )
    def k(uidx_hbm, midx_hbm, uemb_hbm, memb_hbm, ubias_hbm, mbias_hbm,
          out_hbm, uidx_v, midx_v, urows_v, mrows_v, ub_v, mb_v, out_v, sem):
        wid = lax.axis_index("s") * _NC + lax.axis_index("c")
        pltpu.sync_copy(uidx_hbm.at[pl.ds(wid * n_chunks, n_chunks)], uidx_v)
        pltpu.sync_copy(midx_hbm.at[pl.ds(wid * n_chunks, n_chunks)], midx_v)
        copies = []
        for j in range(n_chunks):
            s = j * _CHUNK
            copies.append(pltpu.async_copy(
                uemb_hbm.at[uidx_v.at[j]], urows_v.at[pl.ds(s, _CHUNK)], sem))
            copies.append(pltpu.async_copy(
                memb_hbm.at[midx_v.at[j]], mrows_v.at[pl.ds(s, _CHUNK)], sem))
            copies.append(pltpu.async_copy(
                ubias_hbm.at[uidx_v.at[j]], ub_v.at[pl.ds(s, _CHUNK)], sem))
            copies.append(pltpu.async_copy(
                mbias_hbm.at[midx_v.at[j]], mb_v.at[pl.ds(s, _CHUNK)], sem))
        for c in copies:
            c.wait()

        lanes = lax.iota(jnp.int32, _NL)

        def group(g, carry):
            rows = lanes + g * _NL
            acc = ub_v[pl.ds(g * _NL, _NL)] + mb_v[pl.ds(g * _NL, _NL)]
            for e in range(_EMB):
                col = jnp.full((_NL,), e, jnp.int32)
                acc = acc + (plsc.load_gather(urows_v, [rows, col])
                             * plsc.load_gather(mrows_v, [rows, col]))
            out_v[pl.ds(g * _NL, _NL)] = 1.0 / (1.0 + jnp.exp(-acc))
            return carry

        lax.fori_loop(0, n_groups, group, 0)
        pltpu.sync_copy(out_v, out_hbm.at[pl.ds(wid * bpw, bpw)])

    return k


def kernel(inputs, user_emb, user_bias, movie_emb, movie_bias):
    batch = inputs.shape[0]
    uidx = inputs[:, 0].reshape(batch // _CHUNK, _CHUNK)
    midx = inputs[:, 1].reshape(batch // _CHUNK, _CHUNK)
    out = _recnet_sc(batch)(uidx, midx, user_emb, movie_emb,
                            user_bias.reshape(-1), movie_bias.reshape(-1))
    return out.reshape(batch, 1)
